# baseline (speedup 1.0000x reference)
import jax
import jax.numpy as jnp
from jax.experimental import pallas as pl
from jax.experimental.pallas import tpu as pltpu

_CENTER_VAR = 0.1
_SIZE_VAR = 0.2
_NUM_CLASSES = 21
_TOP_K = 200
_IOU_THR = 0.45
_SCORE_THR = 0.01

_N = 20000
_NB = 1024
_NBLK = 20
_NPAD = _NB * _NBLK
_ROWS = 160
_LANES = 128
_KPAD = 256
_BIG = 2 ** 30
_NEG = float("-inf")


def _decode_body(logits_ref, bbox_ref, priors_ref, out_ref):
    nb = pl.program_id(1)
    x = logits_ref[0]
    m = jnp.max(x, axis=-1, keepdims=True)
    s = jnp.sum(jnp.exp(x - m), axis=-1, keepdims=True)
    fg = x[:, 1:]
    mf = jnp.max(fg, axis=-1, keepdims=True)
    best = jnp.exp(mf - m) / s
    cls_iota = jax.lax.broadcasted_iota(jnp.int32, fg.shape, 1)
    lab = jnp.min(jnp.where(fg >= mf, cls_iota, _BIG), axis=-1, keepdims=True) + 1

    row_iota = jax.lax.broadcasted_iota(jnp.int32, (_NB, 1), 0)
    valid = (nb * _NB + row_iota) < _N
    best = jnp.where(valid, best, _NEG)
    labf = jnp.where(valid, lab, 0).astype(jnp.float32)

    bb = bbox_ref[0]
    pr = priors_ref[...]
    cxy = bb[:, :2] * _CENTER_VAR * pr[:, 2:] + pr[:, :2]
    wh = jnp.exp(bb[:, 2:] * _SIZE_VAR) * pr[:, 2:]
    p1 = cxy - wh * 0.5
    p2 = cxy + wh * 0.5
    zero = jnp.zeros((_NB, 2), jnp.float32)
    out_ref[0] = jnp.concatenate([best, labf, p1, p2, zero], axis=1)


def _topk_body(best_ref, vals_ref, idx_ref, scratch):
    scratch[...] = best_ref[...]
    shape = (16, _ROWS, _LANES)
    fiota = (jax.lax.broadcasted_iota(jnp.int32, shape, 1) * _LANES
             + jax.lax.broadcasted_iota(jnp.int32, shape, 2))
    kiota = jax.lax.broadcasted_iota(jnp.int32, (16, 1, _KPAD), 2)

    def body(k, carry):
        va, ia = carry
        cur = scratch[...]
        m = jnp.max(jnp.max(cur, axis=1, keepdims=True), axis=2, keepdims=True)
        sel = jnp.where(cur == m, fiota, _BIG)
        idx = jnp.min(jnp.min(sel, axis=1, keepdims=True), axis=2, keepdims=True)
        va = jnp.where(kiota == k, jnp.broadcast_to(m, (16, 1, _KPAD)), va)
        ia = jnp.where(kiota == k, jnp.broadcast_to(idx, (16, 1, _KPAD)), ia)
        scratch[...] = jnp.where(fiota == idx, _NEG, cur)
        return va, ia

    va0 = jnp.full((16, 1, _KPAD), _NEG, jnp.float32)
    ia0 = jnp.zeros((16, 1, _KPAD), jnp.int32)
    va, ia = jax.lax.fori_loop(0, _TOP_K, body, (va0, ia0))
    vals_ref[...] = va
    idx_ref[...] = ia


def _nms_body(idx_ref, vals_ref, x1_ref, y1_ref, x2_ref, y2_ref, lab_ref,
              boxes_ref, scores_ref, labels_ref, keep_ref,
              cand_rows, supmat):
    lane128 = jax.lax.broadcasted_iota(jnp.int32, (1, _LANES), 1)
    lane256 = jax.lax.broadcasted_iota(jnp.int32, (1, _KPAD), 1)

    def ext(ref, r, c):
        row = ref[0, pl.ds(r, 1), :]
        return jnp.max(jnp.where(lane128 == c, row, _NEG), axis=1, keepdims=True)

    def gather_body(k, carry):
        x1c, y1c, x2c, y2c, labc = carry
        i = idx_ref[0, 0, k]
        r = i // _LANES
        c = i - r * _LANES
        vx1 = ext(x1_ref, r, c)
        vy1 = ext(y1_ref, r, c)
        vx2 = ext(x2_ref, r, c)
        vy2 = ext(y2_ref, r, c)
        vlb = ext(lab_ref, r, c)
        sel = lane256 == k
        x1c = jnp.where(sel, jnp.broadcast_to(vx1, (1, _KPAD)), x1c)
        y1c = jnp.where(sel, jnp.broadcast_to(vy1, (1, _KPAD)), y1c)
        x2c = jnp.where(sel, jnp.broadcast_to(vx2, (1, _KPAD)), x2c)
        y2c = jnp.where(sel, jnp.broadcast_to(vy2, (1, _KPAD)), y2c)
        labc = jnp.where(sel, jnp.broadcast_to(vlb, (1, _KPAD)), labc)
        cand_rows[pl.ds(k, 1), :] = jnp.concatenate([vx1, vy1, vx2, vy2], axis=1)
        return x1c, y1c, x2c, y2c, labc

    z = jnp.zeros((1, _KPAD), jnp.float32)
    x1c, y1c, x2c, y2c, labc = jax.lax.fori_loop(
        0, _TOP_K, gather_body, (z, z, z, z, z))

    colx1 = cand_rows[:, 0:1]
    coly1 = cand_rows[:, 1:2]
    colx2 = cand_rows[:, 2:3]
    coly2 = cand_rows[:, 3:4]

    ix1 = jnp.maximum(colx1, x1c)
    iy1 = jnp.maximum(coly1, y1c)
    ix2 = jnp.minimum(colx2, x2c)
    iy2 = jnp.minimum(coly2, y2c)
    inter = jnp.clip(ix2 - ix1, 0.0) * jnp.clip(iy2 - iy1, 0.0)
    area_col = jnp.clip(colx2 - colx1, 0.0) * jnp.clip(coly2 - coly1, 0.0)
    area_row = jnp.clip(x2c - x1c, 0.0) * jnp.clip(y2c - y1c, 0.0)
    union = area_col + area_row - inter
    iou = inter / jnp.maximum(union, 1e-9)
    supmat[...] = jnp.where(iou > _IOU_THR, 1.0, 0.0)

    vals_row = vals_ref[0]
    keep0 = jnp.where(vals_row > _SCORE_THR, 1.0, 0.0)

    def nms_step(i, keepf):
        rowi = supmat[pl.ds(i, 1), :]
        labi = jnp.max(jnp.where(lane256 == i, labc, _NEG), axis=1, keepdims=True)
        same_i = jnp.where(labc == jnp.broadcast_to(labi, (1, _KPAD)), 1.0, 0.0)
        cansup = rowi * same_i * keepf * jnp.where(lane256 < i, 1.0, 0.0)
        supp = jnp.max(cansup, axis=1, keepdims=True)
        return jnp.where(lane256 == i,
                         keepf * (1.0 - jnp.broadcast_to(supp, (1, _KPAD))),
                         keepf)

    keepf = jax.lax.fori_loop(0, _TOP_K, nms_step, keep0)

    scores_ref[0] = (vals_row * keepf)[:, :_TOP_K]
    labels_ref[0] = (labc * keepf)[:, :_TOP_K].astype(jnp.int32)
    keep_ref[0] = keepf[:, :_TOP_K].astype(jnp.int32)

    def out_body(k, _):
        kv = jnp.max(jnp.where(lane256 == k, keepf, 0.0), axis=1, keepdims=True)
        row = cand_rows[pl.ds(k, 1), :] * jnp.broadcast_to(kv, (1, 4))
        boxes_ref[0, pl.ds(k, 1), :] = row
        return 0

    jax.lax.fori_loop(0, _TOP_K, out_body, 0)


def kernel(cls_logits, bbox_pred, priors):
    B, N, C = cls_logits.shape

    packed = pl.pallas_call(
        _decode_body,
        grid=(B, _NBLK),
        in_specs=[
            pl.BlockSpec((1, _NB, _NUM_CLASSES), lambda b, nb: (b, nb, 0)),
            pl.BlockSpec((1, _NB, 4), lambda b, nb: (b, nb, 0)),
            pl.BlockSpec((_NB, 4), lambda b, nb: (nb, 0)),
        ],
        out_specs=pl.BlockSpec((1, _NB, 8), lambda b, nb: (b, nb, 0)),
        out_shape=jax.ShapeDtypeStruct((B, _NPAD, 8), jnp.float32),
    )(cls_logits, bbox_pred, priors)

    best2d = packed[:, :, 0].reshape(B, _ROWS, _LANES)
    flab = packed[:, :, 1].reshape(B, _ROWS, _LANES)
    fx1 = packed[:, :, 2].reshape(B, _ROWS, _LANES)
    fy1 = packed[:, :, 3].reshape(B, _ROWS, _LANES)
    fx2 = packed[:, :, 4].reshape(B, _ROWS, _LANES)
    fy2 = packed[:, :, 5].reshape(B, _ROWS, _LANES)

    vals, idxs = pl.pallas_call(
        _topk_body,
        in_specs=[pl.BlockSpec(best2d.shape, lambda: (0, 0, 0))],
        out_specs=[
            pl.BlockSpec((B, 1, _KPAD), lambda: (0, 0, 0)),
            pl.BlockSpec((B, 1, _KPAD), lambda: (0, 0, 0)),
        ],
        out_shape=[
            jax.ShapeDtypeStruct((B, 1, _KPAD), jnp.float32),
            jax.ShapeDtypeStruct((B, 1, _KPAD), jnp.int32),
        ],
        scratch_shapes=[pltpu.VMEM((B, _ROWS, _LANES), jnp.float32)],
    )(best2d)

    field_spec = pl.BlockSpec((1, _ROWS, _LANES), lambda b: (b, 0, 0))
    boxes, scores, labels, keep = pl.pallas_call(
        _nms_body,
        grid=(B,),
        in_specs=[
            pl.BlockSpec((1, 1, _KPAD), lambda b: (b, 0, 0),
                         memory_space=pltpu.SMEM),
            pl.BlockSpec((1, 1, _KPAD), lambda b: (b, 0, 0)),
            field_spec, field_spec, field_spec, field_spec, field_spec,
        ],
        out_specs=[
            pl.BlockSpec((1, _TOP_K, 4), lambda b: (b, 0, 0)),
            pl.BlockSpec((1, 1, _TOP_K), lambda b: (b, 0, 0)),
            pl.BlockSpec((1, 1, _TOP_K), lambda b: (b, 0, 0)),
            pl.BlockSpec((1, 1, _TOP_K), lambda b: (b, 0, 0)),
        ],
        out_shape=[
            jax.ShapeDtypeStruct((B, _TOP_K, 4), jnp.float32),
            jax.ShapeDtypeStruct((B, 1, _TOP_K), jnp.float32),
            jax.ShapeDtypeStruct((B, 1, _TOP_K), jnp.int32),
            jax.ShapeDtypeStruct((B, 1, _TOP_K), jnp.int32),
        ],
        scratch_shapes=[
            pltpu.VMEM((_KPAD, 4), jnp.float32),
            pltpu.VMEM((_KPAD, _KPAD), jnp.float32),
        ],
    )(idxs, vals, fx1, fy1, fx2, fy2, flab)

    return (boxes,
            scores.reshape(B, _TOP_K),
            labels.reshape(B, _TOP_K),
            keep.reshape(B, _TOP_K).astype(bool))


# speedup vs baseline: 2.0536x; 2.0536x over previous
import functools

import jax
import jax.numpy as jnp
from jax import lax
from jax.experimental import pallas as pl
from jax.experimental.pallas import tpu as pltpu
from jax.experimental.pallas import tpu_sc as plsc

_CENTER_VAR = 0.1
_SIZE_VAR = 0.2
_NUM_CLASSES = 21
_TOP_K = 200
_IOU_THR = 0.45
_SCORE_THR = 0.01

_B = 16
_N = 20000
_NB = 1024
_NBLK = 20
_NPAD = _NB * _NBLK
_ROWS = 160
_LANES = 128
_KPAD = 256
_CBUF = 320
_HALF = _NPAD // 2
_BIG = 2 ** 30
_NEG = float("-inf")


def _decode_body(logits_ref, bbox_ref, priors_ref, out_ref):
    nb = pl.program_id(1)
    x = logits_ref[0]
    m = jnp.max(x, axis=-1, keepdims=True)
    s = jnp.sum(jnp.exp(x - m), axis=-1, keepdims=True)
    fg = x[:, 1:]
    mf = jnp.max(fg, axis=-1, keepdims=True)
    best = jnp.exp(mf - m) / s
    cls_iota = lax.broadcasted_iota(jnp.int32, fg.shape, 1)
    lab = jnp.min(jnp.where(fg >= mf, cls_iota, _BIG), axis=-1, keepdims=True) + 1

    row_iota = lax.broadcasted_iota(jnp.int32, (_NB, 1), 0)
    valid = (nb * _NB + row_iota) < _N
    best = jnp.where(valid, best, _NEG)
    labf = jnp.where(valid, lab, 0).astype(jnp.float32)

    bb = bbox_ref[0]
    pr = priors_ref[...]
    cxy = bb[:, :2] * _CENTER_VAR * pr[:, 2:] + pr[:, :2]
    wh = jnp.exp(bb[:, 2:] * _SIZE_VAR) * pr[:, 2:]
    p1 = cxy - wh * 0.5
    p2 = cxy + wh * 0.5
    zero = jnp.zeros((_NB, 2), jnp.float32)
    out_ref[0] = jnp.concatenate([best, labf, p1, p2, zero], axis=1)


def _bisect_body(best_ref, thr_ref):
    view = lax.bitcast_convert_type(best_ref[...], jnp.int32)

    def body(_, carry):
        lo, hi = carry
        mid = lo + (hi - lo) // 2
        ge = (view >= mid).astype(jnp.int32)
        cnt = jnp.sum(jnp.sum(ge, axis=1, keepdims=True), axis=2, keepdims=True)
        take = cnt >= _TOP_K
        return jnp.where(take, mid, lo), jnp.where(take, hi, mid)

    lo0 = jnp.zeros((_B, 1, 1), jnp.int32)
    hi0 = jnp.full((_B, 1, 1), 2 ** 31 - 1, jnp.int32)
    lo, _ = lax.fori_loop(0, 31, body, (lo0, hi0))
    thr = lax.bitcast_convert_type(lo, jnp.float32)
    thr_ref[...] = jnp.broadcast_to(thr, (_B, 1, 16))


def _sc_compact_body(best_hbm, thr_hbm, vals_hbm, idx_hbm,
                     chunk_v, sv_v, si_v, thr_v):
    cid = lax.axis_index("c")
    sid = lax.axis_index("s")
    wid = sid * 2 + cid
    b = wid // 2
    h = wid - b * 2

    pltpu.sync_copy(best_hbm.at[b, pl.ds(h * _HALF, _HALF)], chunk_v)
    pltpu.sync_copy(thr_hbm.at[b], thr_v)
    t16 = thr_v[...]

    neg = jnp.full((16,), _NEG, jnp.float32)
    zero = jnp.zeros((16,), jnp.int32)

    def init(j, _):
        sv_v[pl.ds(j * 16, 16)] = neg
        si_v[pl.ds(j * 16, 16)] = zero
        return 0

    lax.fori_loop(0, _CBUF // 16, init, 0)

    lane = lax.iota(jnp.int32, 16)
    base0 = h * _HALF

    one16 = jnp.ones((16,), jnp.int32)
    zero16 = jnp.zeros((16,), jnp.int32)

    def body(i, off):
        v = chunk_v[pl.ds(i * 16, 16)]
        mask = v >= t16
        cnt = jnp.sum(jnp.where(mask, one16, zero16))
        off_c = jnp.minimum(off, _CBUF - 16)
        idxv = base0 + i * 16 + lane
        plsc.store_compressed(sv_v.at[pl.ds(off_c, 16)], v, mask=mask)
        plsc.store_compressed(si_v.at[pl.ds(off_c, 16)], idxv, mask=mask)
        return jnp.minimum(off + cnt, _CBUF - 16)

    lax.fori_loop(0, _HALF // 16, body, jnp.int32(0))

    pltpu.sync_copy(sv_v, vals_hbm.at[b, h])
    pltpu.sync_copy(si_v, idx_hbm.at[b, h])


def _sort_body(sv_ref, si_ref, gv_ref, gi_ref):
    sv = sv_ref[...]
    si = si_ref[...]
    kiota = lax.broadcasted_iota(jnp.int32, (_B, 1, _KPAD), 2)
    biota = lax.broadcasted_iota(jnp.int32, (_B, 1, _KPAD), 0)

    def body(k, carry):
        sv, va, ia = carry
        m = jnp.max(jnp.max(sv, axis=1, keepdims=True), axis=2, keepdims=True)
        sel = jnp.where(sv == m, si, _BIG)
        idx = jnp.min(jnp.min(sel, axis=1, keepdims=True), axis=2, keepdims=True)
        va = jnp.where(kiota == k, jnp.broadcast_to(m, va.shape), va)
        ia = jnp.where(kiota == k, jnp.broadcast_to(idx, ia.shape), ia)
        sv = jnp.where(si == idx, _NEG, sv)
        return sv, va, ia

    va0 = jnp.full((_B, 1, _KPAD), _NEG, jnp.float32)
    ia0 = jnp.zeros((_B, 1, _KPAD), jnp.int32)
    _, va, ia = lax.fori_loop(0, _TOP_K, body, (sv, va0, ia0))
    gv_ref[...] = va
    gi_ref[...] = ia + biota * _NPAD


def _sc_gather_body(packed_hbm, gidx_hbm, cand_hbm, idx_v, rows_v, sem):
    cid = lax.axis_index("c")
    sid = lax.axis_index("s")
    wid = sid * 2 + cid
    b = wid // 2
    h = wid - b * 2

    pltpu.sync_copy(gidx_hbm.at[b, pl.ds(h * 128, 128)], idx_v)
    pltpu.async_copy(packed_hbm.at[idx_v], rows_v, sem).wait()
    pltpu.sync_copy(rows_v, cand_hbm.at[b, pl.ds(h * 128, 128)])


def _nms_body(cand_ref, gv_ref, boxes_ref, scores_ref, labels_ref, keep_ref,
              supmat):
    lane256 = lax.broadcasted_iota(jnp.int32, (_B, 1, _KPAD), 2)
    cand = cand_ref[...]
    labcol = cand[:, :, 1:2]
    x1col = cand[:, :, 2:3]
    y1col = cand[:, :, 3:4]
    x2col = cand[:, :, 4:5]
    y2col = cand[:, :, 5:6]

    def gather_rows(k, carry):
        x1r, y1r, x2r, y2r, labr = carry
        row = cand_ref[:, pl.ds(k, 1), :]
        sel = lane256 == k

        def put(acc, f):
            v = row[:, :, f:f + 1]
            return jnp.where(sel, jnp.broadcast_to(v, acc.shape), acc)

        return (put(x1r, 2), put(y1r, 3), put(x2r, 4), put(y2r, 5),
                put(labr, 1))

    z = jnp.zeros((_B, 1, _KPAD), jnp.float32)
    x1r, y1r, x2r, y2r, labr = lax.fori_loop(
        0, _TOP_K, gather_rows, (z, z, z, z, z))

    ix1 = jnp.maximum(x1col, x1r)
    iy1 = jnp.maximum(y1col, y1r)
    ix2 = jnp.minimum(x2col, x2r)
    iy2 = jnp.minimum(y2col, y2r)
    inter = jnp.clip(ix2 - ix1, 0.0) * jnp.clip(iy2 - iy1, 0.0)
    area_col = jnp.clip(x2col - x1col, 0.0) * jnp.clip(y2col - y1col, 0.0)
    area_row = jnp.clip(x2r - x1r, 0.0) * jnp.clip(y2r - y1r, 0.0)
    union = area_col + area_row - inter
    iou = inter / jnp.maximum(union, 1e-9)
    same = labcol == labr
    supmat[...] = jnp.where((iou > _IOU_THR) & same, 1.0, 0.0)

    gv = gv_ref[...]
    keep0 = jnp.where(gv > _SCORE_THR, 1.0, 0.0)

    def nms_step(i, keepf):
        rowi = supmat[:, pl.ds(i, 1), :]
        cansup = rowi * keepf * jnp.where(lane256 < i, 1.0, 0.0)
        supp = jnp.max(cansup, axis=2, keepdims=True)
        return jnp.where(lane256 == i,
                         keepf * (1.0 - jnp.broadcast_to(supp, keepf.shape)),
                         keepf)

    keepf = lax.fori_loop(0, _TOP_K, nms_step, keep0)

    scores_ref[...] = (gv * keepf)[:, :, :_TOP_K]
    labels_ref[...] = (labr * keepf)[:, :, :_TOP_K].astype(jnp.int32)
    keep_ref[...] = keepf[:, :, :_TOP_K].astype(jnp.int32)

    def out_body(k, _):
        kv = jnp.max(jnp.where(lane256 == k, keepf, 0.0), axis=2, keepdims=True)
        row = cand_ref[:, pl.ds(k, 1), 2:6] * jnp.broadcast_to(kv, (_B, 1, 4))
        boxes_ref[:, pl.ds(k, 1), :] = row
        return 0

    lax.fori_loop(0, _TOP_K, out_body, 0)


def _sc_compact():
    mesh = plsc.VectorSubcoreMesh(core_axis_name="c", subcore_axis_name="s")
    return pl.kernel(
        _sc_compact_body, mesh=mesh,
        compiler_params=pltpu.CompilerParams(needs_layout_passes=False),
        out_type=[jax.ShapeDtypeStruct((_B, 2, _CBUF), jnp.float32),
                  jax.ShapeDtypeStruct((_B, 2, _CBUF), jnp.int32)],
        scratch_types=[pltpu.VMEM((_HALF,), jnp.float32),
                       pltpu.VMEM((_CBUF,), jnp.float32),
                       pltpu.VMEM((_CBUF,), jnp.int32),
                       pltpu.VMEM((16,), jnp.float32)],
    )


def _sc_gather():
    mesh = plsc.VectorSubcoreMesh(core_axis_name="c", subcore_axis_name="s")
    return pl.kernel(
        _sc_gather_body, mesh=mesh,
        compiler_params=pltpu.CompilerParams(needs_layout_passes=False,
                                             use_tc_tiling_on_sc=False),
        out_type=jax.ShapeDtypeStruct((_B, _KPAD, 8), jnp.float32),
        scratch_types=[pltpu.VMEM((128,), jnp.int32),
                       pltpu.VMEM((128, 8), jnp.float32),
                       pltpu.SemaphoreType.DMA],
    )


def kernel(cls_logits, bbox_pred, priors):
    B, N, C = cls_logits.shape

    packed = pl.pallas_call(
        _decode_body,
        grid=(B, _NBLK),
        in_specs=[
            pl.BlockSpec((1, _NB, _NUM_CLASSES), lambda b, nb: (b, nb, 0)),
            pl.BlockSpec((1, _NB, 4), lambda b, nb: (b, nb, 0)),
            pl.BlockSpec((_NB, 4), lambda b, nb: (nb, 0)),
        ],
        out_specs=pl.BlockSpec((1, _NB, 8), lambda b, nb: (b, nb, 0)),
        out_shape=jax.ShapeDtypeStruct((B, _NPAD, 8), jnp.float32),
    )(cls_logits, bbox_pred, priors)

    best2d = packed[:, :, 0].reshape(B, _ROWS, _LANES)

    thr = pl.pallas_call(
        _bisect_body,
        in_specs=[pl.BlockSpec((B, _ROWS, _LANES), lambda: (0, 0, 0))],
        out_specs=pl.BlockSpec((B, 1, 16), lambda: (0, 0, 0)),
        out_shape=jax.ShapeDtypeStruct((B, 1, 16), jnp.float32),
    )(best2d)

    sv, si = _sc_compact()(best2d.reshape(B, _NPAD), thr.reshape(B, 16))

    gv, gi = pl.pallas_call(
        _sort_body,
        in_specs=[
            pl.BlockSpec((B, 1, 2 * _CBUF), lambda: (0, 0, 0)),
            pl.BlockSpec((B, 1, 2 * _CBUF), lambda: (0, 0, 0)),
        ],
        out_specs=[
            pl.BlockSpec((B, 1, _KPAD), lambda: (0, 0, 0)),
            pl.BlockSpec((B, 1, _KPAD), lambda: (0, 0, 0)),
        ],
        out_shape=[
            jax.ShapeDtypeStruct((B, 1, _KPAD), jnp.float32),
            jax.ShapeDtypeStruct((B, 1, _KPAD), jnp.int32),
        ],
    )(sv.reshape(B, 1, 2 * _CBUF), si.reshape(B, 1, 2 * _CBUF))

    cand = _sc_gather()(packed.reshape(B * _NPAD, 8), gi.reshape(B, _KPAD))

    boxes, scores, labels, keep = pl.pallas_call(
        _nms_body,
        in_specs=[
            pl.BlockSpec((B, _KPAD, 8), lambda: (0, 0, 0)),
            pl.BlockSpec((B, 1, _KPAD), lambda: (0, 0, 0)),
        ],
        out_specs=[
            pl.BlockSpec((B, _TOP_K, 4), lambda: (0, 0, 0)),
            pl.BlockSpec((B, 1, _TOP_K), lambda: (0, 0, 0)),
            pl.BlockSpec((B, 1, _TOP_K), lambda: (0, 0, 0)),
            pl.BlockSpec((B, 1, _TOP_K), lambda: (0, 0, 0)),
        ],
        out_shape=[
            jax.ShapeDtypeStruct((B, _TOP_K, 4), jnp.float32),
            jax.ShapeDtypeStruct((B, 1, _TOP_K), jnp.float32),
            jax.ShapeDtypeStruct((B, 1, _TOP_K), jnp.int32),
            jax.ShapeDtypeStruct((B, 1, _TOP_K), jnp.int32),
        ],
        scratch_shapes=[pltpu.VMEM((_B, _KPAD, _KPAD), jnp.float32)],
    )(cand, gv)

    return (boxes,
            scores.reshape(B, _TOP_K),
            labels.reshape(B, _TOP_K),
            keep.reshape(B, _TOP_K).astype(bool))


# speedup vs baseline: 2.7879x; 1.3576x over previous
import functools

import jax
import jax.numpy as jnp
from jax import lax
from jax.experimental import pallas as pl
from jax.experimental.pallas import tpu as pltpu
from jax.experimental.pallas import tpu_sc as plsc

_CENTER_VAR = 0.1
_SIZE_VAR = 0.2
_NUM_CLASSES = 21
_TOP_K = 200
_IOU_THR = 0.45
_SCORE_THR = 0.01

_B = 16
_N = 20000
_NB = 1024
_NBLK = 20
_NPAD = _NB * _NBLK
_ROWS = 160
_LANES = 128
_KPAD = 256
_CBUF = 320
_HALF = _NPAD // 2
_BIG = 2 ** 30
_NEG = float("-inf")


def _decode_body(logits_ref, bbox_ref, priors_ref, packed_ref, best_ref):
    nb = pl.program_id(1)
    x = logits_ref[0]
    xt = jnp.transpose(x, (1, 0))
    m = jnp.max(xt, axis=0, keepdims=True)
    e = jnp.exp(xt - m)
    s = jnp.sum(e, axis=0, keepdims=True)
    efg = e[1:, :]
    eb = jnp.max(efg, axis=0, keepdims=True)
    best = eb / s
    cls_iota = lax.broadcasted_iota(jnp.int32, efg.shape, 0)
    lab = jnp.min(jnp.where(efg >= eb, cls_iota, _BIG), axis=0, keepdims=True) + 1

    lane_iota = lax.broadcasted_iota(jnp.int32, (1, _NB), 1)
    valid = (nb * _NB + lane_iota) < _N
    best = jnp.where(valid, best, _NEG)
    labf = jnp.where(valid, lab, 0).astype(jnp.float32)

    bt = jnp.transpose(bbox_ref[0], (1, 0))
    pt = jnp.transpose(priors_ref[...], (1, 0))
    cx = bt[0:1] * _CENTER_VAR * pt[2:3] + pt[0:1]
    cy = bt[1:2] * _CENTER_VAR * pt[3:4] + pt[1:2]
    w = jnp.exp(bt[2:3] * _SIZE_VAR) * pt[2:3]
    h = jnp.exp(bt[3:4] * _SIZE_VAR) * pt[3:4]
    x1 = cx - w * 0.5
    y1 = cy - h * 0.5
    x2 = cx + w * 0.5
    y2 = cy + h * 0.5
    z = jnp.zeros((1, _NB), jnp.float32)
    stack = jnp.concatenate([best, labf, x1, y1, x2, y2, z, z], axis=0)
    packed_ref[0] = jnp.transpose(stack, (1, 0))
    best_ref[0, 0] = best


def _bisect_body(best_ref, thr_ref):
    view = lax.bitcast_convert_type(best_ref[...], jnp.int32)

    def body(_, carry):
        lo, hi = carry
        mid = lo + (hi - lo) // 2
        ge = (view >= mid).astype(jnp.int32)
        cnt = jnp.sum(jnp.sum(ge, axis=1, keepdims=True), axis=2, keepdims=True)
        take = cnt >= _TOP_K
        return jnp.where(take, mid, lo), jnp.where(take, hi, mid)

    lo0 = jnp.zeros((_B, 1, 1), jnp.int32)
    hi0 = jnp.full((_B, 1, 1), 2 ** 31 - 1, jnp.int32)
    lo, _ = lax.fori_loop(0, 31, body, (lo0, hi0))
    thr = lax.bitcast_convert_type(lo, jnp.float32)
    thr_ref[...] = jnp.broadcast_to(thr, (_B, 1, 16))


def _sc_compact_body(best_hbm, thr_hbm, vals_hbm, idx_hbm,
                     chunk_v, sv_v, si_v, thr_v):
    cid = lax.axis_index("c")
    sid = lax.axis_index("s")
    wid = sid * 2 + cid
    b = wid // 2
    h = wid - b * 2

    pltpu.sync_copy(best_hbm.at[b, pl.ds(h * _HALF, _HALF)], chunk_v)
    pltpu.sync_copy(thr_hbm.at[b], thr_v)
    t16 = thr_v[...]

    neg = jnp.full((16,), _NEG, jnp.float32)
    zero = jnp.zeros((16,), jnp.int32)

    def init(j, _):
        sv_v[pl.ds(j * 16, 16)] = neg
        si_v[pl.ds(j * 16, 16)] = zero
        return 0

    lax.fori_loop(0, _CBUF // 16, init, 0)

    lane = lax.iota(jnp.int32, 16)
    base0 = h * _HALF

    one16 = jnp.ones((16,), jnp.int32)
    zero16 = jnp.zeros((16,), jnp.int32)

    def body(i, off):
        v = chunk_v[pl.ds(i * 16, 16)]
        mask = v >= t16
        cnt = jnp.sum(jnp.where(mask, one16, zero16))
        off_c = jnp.minimum(off, _CBUF - 16)
        idxv = base0 + i * 16 + lane
        plsc.store_compressed(sv_v.at[pl.ds(off_c, 16)], v, mask=mask)
        plsc.store_compressed(si_v.at[pl.ds(off_c, 16)], idxv, mask=mask)
        return jnp.minimum(off + cnt, _CBUF - 16)

    lax.fori_loop(0, _HALF // 16, body, jnp.int32(0))

    pltpu.sync_copy(sv_v, vals_hbm.at[b, h])
    pltpu.sync_copy(si_v, idx_hbm.at[b, h])


def _sort_body(sv_ref, si_ref, gv_ref, gi_ref):
    sv = sv_ref[...]
    si = si_ref[...]
    kiota = lax.broadcasted_iota(jnp.int32, (_B, 1, _KPAD), 2)
    biota = lax.broadcasted_iota(jnp.int32, (_B, 1, _KPAD), 0)

    def body(k, carry):
        sv, va, ia = carry
        m = jnp.max(jnp.max(sv, axis=1, keepdims=True), axis=2, keepdims=True)
        sel = jnp.where(sv == m, si, _BIG)
        idx = jnp.min(jnp.min(sel, axis=1, keepdims=True), axis=2, keepdims=True)
        va = jnp.where(kiota == k, jnp.broadcast_to(m, va.shape), va)
        ia = jnp.where(kiota == k, jnp.broadcast_to(idx, ia.shape), ia)
        sv = jnp.where(si == idx, _NEG, sv)
        return sv, va, ia

    va0 = jnp.full((_B, 1, _KPAD), _NEG, jnp.float32)
    ia0 = jnp.zeros((_B, 1, _KPAD), jnp.int32)
    _, va, ia = lax.fori_loop(0, _TOP_K, body, (sv, va0, ia0))
    gv_ref[...] = va
    gi_ref[...] = ia + biota * _NPAD


def _sc_gather_body(packed_hbm, gidx_hbm, cand_hbm, idx_v, rows_v, sem):
    cid = lax.axis_index("c")
    sid = lax.axis_index("s")
    wid = sid * 2 + cid
    b = wid // 2
    h = wid - b * 2

    pltpu.sync_copy(gidx_hbm.at[b, pl.ds(h * 128, 128)], idx_v)
    pltpu.async_copy(packed_hbm.at[idx_v], rows_v, sem).wait()
    pltpu.sync_copy(rows_v, cand_hbm.at[b, pl.ds(h * 128, 128)])


def _nms_body(cand_ref, gv_ref, boxes_ref, scores_ref, labels_ref, keep_ref,
              supmat):
    lane256 = lax.broadcasted_iota(jnp.int32, (_B, 1, _KPAD), 2)
    cand = cand_ref[...]
    labcol = cand[:, :, 1:2]
    x1col = cand[:, :, 2:3]
    y1col = cand[:, :, 3:4]
    x2col = cand[:, :, 4:5]
    y2col = cand[:, :, 5:6]

    def gather_rows(k, carry):
        x1r, y1r, x2r, y2r, labr = carry
        row = cand_ref[:, pl.ds(k, 1), :]
        sel = lane256 == k

        def put(acc, f):
            v = row[:, :, f:f + 1]
            return jnp.where(sel, jnp.broadcast_to(v, acc.shape), acc)

        return (put(x1r, 2), put(y1r, 3), put(x2r, 4), put(y2r, 5),
                put(labr, 1))

    z = jnp.zeros((_B, 1, _KPAD), jnp.float32)
    x1r, y1r, x2r, y2r, labr = lax.fori_loop(
        0, _TOP_K, gather_rows, (z, z, z, z, z))

    ix1 = jnp.maximum(x1col, x1r)
    iy1 = jnp.maximum(y1col, y1r)
    ix2 = jnp.minimum(x2col, x2r)
    iy2 = jnp.minimum(y2col, y2r)
    inter = jnp.clip(ix2 - ix1, 0.0) * jnp.clip(iy2 - iy1, 0.0)
    area_col = jnp.clip(x2col - x1col, 0.0) * jnp.clip(y2col - y1col, 0.0)
    area_row = jnp.clip(x2r - x1r, 0.0) * jnp.clip(y2r - y1r, 0.0)
    union = area_col + area_row - inter
    iou = inter / jnp.maximum(union, 1e-9)
    same = labcol == labr
    supmat[...] = jnp.where((iou > _IOU_THR) & same, 1.0, 0.0)

    gv = gv_ref[...]
    keep0 = jnp.where(gv > _SCORE_THR, 1.0, 0.0)

    def nms_step(i, keepf):
        rowi = supmat[:, pl.ds(i, 1), :]
        cansup = rowi * keepf * jnp.where(lane256 < i, 1.0, 0.0)
        supp = jnp.max(cansup, axis=2, keepdims=True)
        return jnp.where(lane256 == i,
                         keepf * (1.0 - jnp.broadcast_to(supp, keepf.shape)),
                         keepf)

    keepf = lax.fori_loop(0, _TOP_K, nms_step, keep0)

    scores_ref[...] = (gv * keepf)[:, :, :_TOP_K]
    labels_ref[...] = (labr * keepf)[:, :, :_TOP_K].astype(jnp.int32)
    keep_ref[...] = keepf[:, :, :_TOP_K].astype(jnp.int32)

    def out_body(k, _):
        kv = jnp.max(jnp.where(lane256 == k, keepf, 0.0), axis=2, keepdims=True)
        row = cand_ref[:, pl.ds(k, 1), 2:6] * jnp.broadcast_to(kv, (_B, 1, 4))
        boxes_ref[:, pl.ds(k, 1), :] = row
        return 0

    lax.fori_loop(0, _TOP_K, out_body, 0)


def _sc_compact():
    mesh = plsc.VectorSubcoreMesh(core_axis_name="c", subcore_axis_name="s")
    return pl.kernel(
        _sc_compact_body, mesh=mesh,
        compiler_params=pltpu.CompilerParams(needs_layout_passes=False),
        out_type=[jax.ShapeDtypeStruct((_B, 2, _CBUF), jnp.float32),
                  jax.ShapeDtypeStruct((_B, 2, _CBUF), jnp.int32)],
        scratch_types=[pltpu.VMEM((_HALF,), jnp.float32),
                       pltpu.VMEM((_CBUF,), jnp.float32),
                       pltpu.VMEM((_CBUF,), jnp.int32),
                       pltpu.VMEM((16,), jnp.float32)],
    )


def _sc_gather():
    mesh = plsc.VectorSubcoreMesh(core_axis_name="c", subcore_axis_name="s")
    return pl.kernel(
        _sc_gather_body, mesh=mesh,
        compiler_params=pltpu.CompilerParams(needs_layout_passes=False,
                                             use_tc_tiling_on_sc=False),
        out_type=jax.ShapeDtypeStruct((_B, _KPAD, 8), jnp.float32),
        scratch_types=[pltpu.VMEM((128,), jnp.int32),
                       pltpu.VMEM((128, 8), jnp.float32),
                       pltpu.SemaphoreType.DMA],
    )


def kernel(cls_logits, bbox_pred, priors):
    B, N, C = cls_logits.shape

    packed, bestp = pl.pallas_call(
        _decode_body,
        grid=(B, _NBLK),
        in_specs=[
            pl.BlockSpec((1, _NB, _NUM_CLASSES), lambda b, nb: (b, nb, 0)),
            pl.BlockSpec((1, _NB, 4), lambda b, nb: (b, nb, 0)),
            pl.BlockSpec((_NB, 4), lambda b, nb: (nb, 0)),
        ],
        out_specs=[
            pl.BlockSpec((1, _NB, 8), lambda b, nb: (b, nb, 0)),
            pl.BlockSpec((1, 1, 1, _NB), lambda b, nb: (b, nb, 0, 0)),
        ],
        out_shape=[
            jax.ShapeDtypeStruct((B, _NPAD, 8), jnp.float32),
            jax.ShapeDtypeStruct((B, _NBLK, 1, _NB), jnp.float32),
        ],
    )(cls_logits, bbox_pred, priors)

    best2d = bestp.reshape(B, _ROWS, _LANES)

    thr = pl.pallas_call(
        _bisect_body,
        in_specs=[pl.BlockSpec((B, _ROWS, _LANES), lambda: (0, 0, 0))],
        out_specs=pl.BlockSpec((B, 1, 16), lambda: (0, 0, 0)),
        out_shape=jax.ShapeDtypeStruct((B, 1, 16), jnp.float32),
    )(best2d)

    sv, si = _sc_compact()(best2d.reshape(B, _NPAD), thr.reshape(B, 16))

    gv, gi = pl.pallas_call(
        _sort_body,
        in_specs=[
            pl.BlockSpec((B, 1, 2 * _CBUF), lambda: (0, 0, 0)),
            pl.BlockSpec((B, 1, 2 * _CBUF), lambda: (0, 0, 0)),
        ],
        out_specs=[
            pl.BlockSpec((B, 1, _KPAD), lambda: (0, 0, 0)),
            pl.BlockSpec((B, 1, _KPAD), lambda: (0, 0, 0)),
        ],
        out_shape=[
            jax.ShapeDtypeStruct((B, 1, _KPAD), jnp.float32),
            jax.ShapeDtypeStruct((B, 1, _KPAD), jnp.int32),
        ],
    )(sv.reshape(B, 1, 2 * _CBUF), si.reshape(B, 1, 2 * _CBUF))

    cand = _sc_gather()(packed.reshape(B * _NPAD, 8), gi.reshape(B, _KPAD))

    boxes, scores, labels, keep = pl.pallas_call(
        _nms_body,
        in_specs=[
            pl.BlockSpec((B, _KPAD, 8), lambda: (0, 0, 0)),
            pl.BlockSpec((B, 1, _KPAD), lambda: (0, 0, 0)),
        ],
        out_specs=[
            pl.BlockSpec((B, _TOP_K, 4), lambda: (0, 0, 0)),
            pl.BlockSpec((B, 1, _TOP_K), lambda: (0, 0, 0)),
            pl.BlockSpec((B, 1, _TOP_K), lambda: (0, 0, 0)),
            pl.BlockSpec((B, 1, _TOP_K), lambda: (0, 0, 0)),
        ],
        out_shape=[
            jax.ShapeDtypeStruct((B, _TOP_K, 4), jnp.float32),
            jax.ShapeDtypeStruct((B, 1, _TOP_K), jnp.float32),
            jax.ShapeDtypeStruct((B, 1, _TOP_K), jnp.int32),
            jax.ShapeDtypeStruct((B, 1, _TOP_K), jnp.int32),
        ],
        scratch_shapes=[pltpu.VMEM((_B, _KPAD, _KPAD), jnp.float32)],
    )(cand, gv)

    return (boxes,
            scores.reshape(B, _TOP_K),
            labels.reshape(B, _TOP_K),
            keep.reshape(B, _TOP_K).astype(bool))


# speedup vs baseline: 4.1792x; 1.4991x over previous
import functools

import jax
import jax.numpy as jnp
from jax import lax
from jax.experimental import pallas as pl
from jax.experimental.pallas import tpu as pltpu
from jax.experimental.pallas import tpu_sc as plsc

_CENTER_VAR = 0.1
_SIZE_VAR = 0.2
_NUM_CLASSES = 21
_TOP_K = 200
_IOU_THR = 0.45
_SCORE_THR = 0.01

_B = 16
_N = 20000
_NB = 1024
_NBLK = 20
_NPAD = _NB * _NBLK
_ROWS = 160
_LANES = 128
_KPAD = 256
_CBUF = 320
_HALF = _NPAD // 2
_BIG = 2 ** 30
_NEG = float("-inf")


def _decode_body(logits_ref, bbox_ref, priors_ref, packed_ref, best_ref):
    nb = pl.program_id(1)
    xt = logits_ref[0]
    m = jnp.max(xt, axis=0, keepdims=True)
    e = jnp.exp(xt - m)
    s = jnp.sum(e, axis=0, keepdims=True)
    efg = e[1:, :]
    eb = jnp.max(efg, axis=0, keepdims=True)
    best = eb / s
    cls_iota = lax.broadcasted_iota(jnp.int32, efg.shape, 0)
    lab = jnp.min(jnp.where(efg >= eb, cls_iota, _BIG), axis=0, keepdims=True) + 1

    lane_iota = lax.broadcasted_iota(jnp.int32, (1, _NB), 1)
    valid = (nb * _NB + lane_iota) < _N
    best = jnp.where(valid, best, _NEG)
    labf = jnp.where(valid, lab, 0).astype(jnp.float32)

    bt = bbox_ref[0]
    pt = priors_ref[...]
    cx = bt[0:1] * _CENTER_VAR * pt[2:3] + pt[0:1]
    cy = bt[1:2] * _CENTER_VAR * pt[3:4] + pt[1:2]
    w = jnp.exp(bt[2:3] * _SIZE_VAR) * pt[2:3]
    h = jnp.exp(bt[3:4] * _SIZE_VAR) * pt[3:4]
    x1 = cx - w * 0.5
    y1 = cy - h * 0.5
    x2 = cx + w * 0.5
    y2 = cy + h * 0.5
    z = jnp.zeros((1, _NB), jnp.float32)
    stack = jnp.concatenate([best, labf, x1, y1, x2, y2, z, z], axis=0)
    packed_ref[0] = jnp.transpose(stack, (1, 0))
    best_ref[0, 0] = best


def _bisect_body(best_ref, thr_ref):
    view = lax.bitcast_convert_type(best_ref[...], jnp.int32)

    def body(_, carry):
        lo, hi = carry
        mid = lo + (hi - lo) // 2
        ge = (view >= mid).astype(jnp.int32)
        cnt = jnp.sum(jnp.sum(ge, axis=1, keepdims=True), axis=2, keepdims=True)
        take = cnt >= _TOP_K
        return jnp.where(take, mid, lo), jnp.where(take, hi, mid)

    lo0 = jnp.zeros((_B, 1, 1), jnp.int32)
    hi0 = jnp.full((_B, 1, 1), 2 ** 31 - 1, jnp.int32)
    lo, _ = lax.fori_loop(0, 31, body, (lo0, hi0))
    thr = lax.bitcast_convert_type(lo, jnp.float32)
    thr_ref[...] = jnp.broadcast_to(thr, (_B, 1, 16))


def _sc_compact_body(best_hbm, thr_hbm, vals_hbm, idx_hbm,
                     chunk_v, sv_v, si_v, thr_v):
    cid = lax.axis_index("c")
    sid = lax.axis_index("s")
    wid = sid * 2 + cid
    b = wid // 2
    h = wid - b * 2

    pltpu.sync_copy(best_hbm.at[b, pl.ds(h * _HALF, _HALF)], chunk_v)
    pltpu.sync_copy(thr_hbm.at[b], thr_v)
    t16 = thr_v[...]

    neg = jnp.full((16,), _NEG, jnp.float32)
    zero = jnp.zeros((16,), jnp.int32)

    def init(j, _):
        sv_v[pl.ds(j * 16, 16)] = neg
        si_v[pl.ds(j * 16, 16)] = zero
        return 0

    lax.fori_loop(0, _CBUF // 16, init, 0)

    lane = lax.iota(jnp.int32, 16)
    base0 = h * _HALF

    one16 = jnp.ones((16,), jnp.int32)
    zero16 = jnp.zeros((16,), jnp.int32)

    def body(i, off):
        v = chunk_v[pl.ds(i * 16, 16)]
        mask = v >= t16
        cnt = jnp.sum(jnp.where(mask, one16, zero16))
        off_c = jnp.minimum(off, _CBUF - 16)
        idxv = base0 + i * 16 + lane
        plsc.store_compressed(sv_v.at[pl.ds(off_c, 16)], v, mask=mask)
        plsc.store_compressed(si_v.at[pl.ds(off_c, 16)], idxv, mask=mask)
        return jnp.minimum(off + cnt, _CBUF - 16)

    lax.fori_loop(0, _HALF // 16, body, jnp.int32(0))

    pltpu.sync_copy(sv_v, vals_hbm.at[b, h])
    pltpu.sync_copy(si_v, idx_hbm.at[b, h])


def _sort_body(sv_ref, si_ref, gv_ref, gi_ref):
    sv = sv_ref[...]
    si = si_ref[...]
    kiota = lax.broadcasted_iota(jnp.int32, (_B, 1, _KPAD), 2)
    biota = lax.broadcasted_iota(jnp.int32, (_B, 1, _KPAD), 0)

    def body(k, carry):
        sv, va, ia = carry
        m = jnp.max(jnp.max(sv, axis=1, keepdims=True), axis=2, keepdims=True)
        sel = jnp.where(sv == m, si, _BIG)
        idx = jnp.min(jnp.min(sel, axis=1, keepdims=True), axis=2, keepdims=True)
        va = jnp.where(kiota == k, jnp.broadcast_to(m, va.shape), va)
        ia = jnp.where(kiota == k, jnp.broadcast_to(idx, ia.shape), ia)
        sv = jnp.where(si == idx, _NEG, sv)
        return sv, va, ia

    va0 = jnp.full((_B, 1, _KPAD), _NEG, jnp.float32)
    ia0 = jnp.zeros((_B, 1, _KPAD), jnp.int32)
    _, va, ia = lax.fori_loop(0, _TOP_K, body, (sv, va0, ia0))
    gv_ref[...] = va
    gi_ref[...] = ia + biota * _NPAD


def _sc_gather_body(packed_hbm, gidx_hbm, cand_hbm, idx_v, rows_v, sem):
    cid = lax.axis_index("c")
    sid = lax.axis_index("s")
    wid = sid * 2 + cid
    b = wid // 2
    h = wid - b * 2

    pltpu.sync_copy(gidx_hbm.at[b, pl.ds(h * 128, 128)], idx_v)
    pltpu.async_copy(packed_hbm.at[idx_v], rows_v, sem).wait()
    pltpu.sync_copy(rows_v, cand_hbm.at[b, pl.ds(h * 128, 128)])


def _nms_body(cand_ref, gv_ref, boxes_ref, scores_ref, labels_ref, keep_ref,
              supmat):
    lane256 = lax.broadcasted_iota(jnp.int32, (_B, 1, _KPAD), 2)
    cand = cand_ref[...]
    labcol = cand[:, :, 1:2]
    x1col = cand[:, :, 2:3]
    y1col = cand[:, :, 3:4]
    x2col = cand[:, :, 4:5]
    y2col = cand[:, :, 5:6]

    def gather_rows(k, carry):
        x1r, y1r, x2r, y2r, labr = carry
        row = cand_ref[:, pl.ds(k, 1), :]
        sel = lane256 == k

        def put(acc, f):
            v = row[:, :, f:f + 1]
            return jnp.where(sel, jnp.broadcast_to(v, acc.shape), acc)

        return (put(x1r, 2), put(y1r, 3), put(x2r, 4), put(y2r, 5),
                put(labr, 1))

    z = jnp.zeros((_B, 1, _KPAD), jnp.float32)
    x1r, y1r, x2r, y2r, labr = lax.fori_loop(
        0, _TOP_K, gather_rows, (z, z, z, z, z))

    ix1 = jnp.maximum(x1col, x1r)
    iy1 = jnp.maximum(y1col, y1r)
    ix2 = jnp.minimum(x2col, x2r)
    iy2 = jnp.minimum(y2col, y2r)
    inter = jnp.clip(ix2 - ix1, 0.0) * jnp.clip(iy2 - iy1, 0.0)
    area_col = jnp.clip(x2col - x1col, 0.0) * jnp.clip(y2col - y1col, 0.0)
    area_row = jnp.clip(x2r - x1r, 0.0) * jnp.clip(y2r - y1r, 0.0)
    union = area_col + area_row - inter
    iou = inter / jnp.maximum(union, 1e-9)
    same = labcol == labr
    supmat[...] = jnp.where((iou > _IOU_THR) & same, 1.0, 0.0)

    gv = gv_ref[...]
    keep0 = jnp.where(gv > _SCORE_THR, 1.0, 0.0)

    def nms_step(i, keepf):
        rowi = supmat[:, pl.ds(i, 1), :]
        cansup = rowi * keepf * jnp.where(lane256 < i, 1.0, 0.0)
        supp = jnp.max(cansup, axis=2, keepdims=True)
        return jnp.where(lane256 == i,
                         keepf * (1.0 - jnp.broadcast_to(supp, keepf.shape)),
                         keepf)

    keepf = lax.fori_loop(0, _TOP_K, nms_step, keep0)

    scores_ref[...] = (gv * keepf)[:, :, :_TOP_K]
    labels_ref[...] = (labr * keepf)[:, :, :_TOP_K].astype(jnp.int32)
    keep_ref[...] = keepf[:, :, :_TOP_K].astype(jnp.int32)

    def out_body(k, _):
        kv = jnp.max(jnp.where(lane256 == k, keepf, 0.0), axis=2, keepdims=True)
        row = cand_ref[:, pl.ds(k, 1), 2:6] * jnp.broadcast_to(kv, (_B, 1, 4))
        boxes_ref[:, pl.ds(k, 1), :] = row
        return 0

    lax.fori_loop(0, _TOP_K, out_body, 0)


def _sc_compact():
    mesh = plsc.VectorSubcoreMesh(core_axis_name="c", subcore_axis_name="s")
    return pl.kernel(
        _sc_compact_body, mesh=mesh,
        compiler_params=pltpu.CompilerParams(needs_layout_passes=False),
        out_type=[jax.ShapeDtypeStruct((_B, 2, _CBUF), jnp.float32),
                  jax.ShapeDtypeStruct((_B, 2, _CBUF), jnp.int32)],
        scratch_types=[pltpu.VMEM((_HALF,), jnp.float32),
                       pltpu.VMEM((_CBUF,), jnp.float32),
                       pltpu.VMEM((_CBUF,), jnp.int32),
                       pltpu.VMEM((16,), jnp.float32)],
    )


def _sc_gather():
    mesh = plsc.VectorSubcoreMesh(core_axis_name="c", subcore_axis_name="s")
    return pl.kernel(
        _sc_gather_body, mesh=mesh,
        compiler_params=pltpu.CompilerParams(needs_layout_passes=False,
                                             use_tc_tiling_on_sc=False),
        out_type=jax.ShapeDtypeStruct((_B, _KPAD, 8), jnp.float32),
        scratch_types=[pltpu.VMEM((128,), jnp.int32),
                       pltpu.VMEM((128, 8), jnp.float32),
                       pltpu.SemaphoreType.DMA],
    )


def kernel(cls_logits, bbox_pred, priors):
    B, N, C = cls_logits.shape

    logits_t = jnp.swapaxes(cls_logits, 1, 2)
    bbox_t = jnp.swapaxes(bbox_pred, 1, 2)
    priors_t = jnp.transpose(priors, (1, 0))

    packed, bestp = pl.pallas_call(
        _decode_body,
        grid=(B, _NBLK),
        in_specs=[
            pl.BlockSpec((1, _NUM_CLASSES, _NB), lambda b, nb: (b, 0, nb)),
            pl.BlockSpec((1, 4, _NB), lambda b, nb: (b, 0, nb)),
            pl.BlockSpec((4, _NB), lambda b, nb: (0, nb)),
        ],
        out_specs=[
            pl.BlockSpec((1, _NB, 8), lambda b, nb: (b, nb, 0)),
            pl.BlockSpec((1, 1, 1, _NB), lambda b, nb: (b, nb, 0, 0)),
        ],
        out_shape=[
            jax.ShapeDtypeStruct((B, _NPAD, 8), jnp.float32),
            jax.ShapeDtypeStruct((B, _NBLK, 1, _NB), jnp.float32),
        ],
    )(logits_t, bbox_t, priors_t)

    best2d = bestp.reshape(B, _ROWS, _LANES)

    thr = pl.pallas_call(
        _bisect_body,
        in_specs=[pl.BlockSpec((B, _ROWS, _LANES), lambda: (0, 0, 0))],
        out_specs=pl.BlockSpec((B, 1, 16), lambda: (0, 0, 0)),
        out_shape=jax.ShapeDtypeStruct((B, 1, 16), jnp.float32),
    )(best2d)

    sv, si = _sc_compact()(best2d.reshape(B, _NPAD), thr.reshape(B, 16))

    gv, gi = pl.pallas_call(
        _sort_body,
        in_specs=[
            pl.BlockSpec((B, 1, 2 * _CBUF), lambda: (0, 0, 0)),
            pl.BlockSpec((B, 1, 2 * _CBUF), lambda: (0, 0, 0)),
        ],
        out_specs=[
            pl.BlockSpec((B, 1, _KPAD), lambda: (0, 0, 0)),
            pl.BlockSpec((B, 1, _KPAD), lambda: (0, 0, 0)),
        ],
        out_shape=[
            jax.ShapeDtypeStruct((B, 1, _KPAD), jnp.float32),
            jax.ShapeDtypeStruct((B, 1, _KPAD), jnp.int32),
        ],
    )(sv.reshape(B, 1, 2 * _CBUF), si.reshape(B, 1, 2 * _CBUF))

    cand = _sc_gather()(packed.reshape(B * _NPAD, 8), gi.reshape(B, _KPAD))

    boxes, scores, labels, keep = pl.pallas_call(
        _nms_body,
        in_specs=[
            pl.BlockSpec((B, _KPAD, 8), lambda: (0, 0, 0)),
            pl.BlockSpec((B, 1, _KPAD), lambda: (0, 0, 0)),
        ],
        out_specs=[
            pl.BlockSpec((B, _TOP_K, 4), lambda: (0, 0, 0)),
            pl.BlockSpec((B, 1, _TOP_K), lambda: (0, 0, 0)),
            pl.BlockSpec((B, 1, _TOP_K), lambda: (0, 0, 0)),
            pl.BlockSpec((B, 1, _TOP_K), lambda: (0, 0, 0)),
        ],
        out_shape=[
            jax.ShapeDtypeStruct((B, _TOP_K, 4), jnp.float32),
            jax.ShapeDtypeStruct((B, 1, _TOP_K), jnp.float32),
            jax.ShapeDtypeStruct((B, 1, _TOP_K), jnp.int32),
            jax.ShapeDtypeStruct((B, 1, _TOP_K), jnp.int32),
        ],
        scratch_shapes=[pltpu.VMEM((_B, _KPAD, _KPAD), jnp.float32)],
    )(cand, gv)

    return (boxes,
            scores.reshape(B, _TOP_K),
            labels.reshape(B, _TOP_K),
            keep.reshape(B, _TOP_K).astype(bool))
